# baseline (speedup 1.0000x reference)
import functools
import math

import jax
import jax.numpy as jnp
import numpy as np
from jax import lax
from jax.experimental import pallas as pl
from jax.experimental.pallas import tpu as pltpu
from jax.experimental.pallas import tpu_sc as plsc

VOCAB = 100000
SEQ_LEN = 2048
D_MODEL = 768
BATCH = 4
SCALE = math.sqrt(float(D_MODEL))

NUM_WORKERS = 32
SEQ_PER_W = SEQ_LEN // NUM_WORKERS
LANES = 16
CHUNKS_PER_ROW = D_MODEL // LANES


def _make_pe() -> np.ndarray:
    pos = np.arange(SEQ_LEN, dtype=np.float32)[:, None]
    div = np.exp(
        np.arange(0, D_MODEL, 2, dtype=np.float32)
        * (-math.log(10000.0) / D_MODEL)
    )
    pe = np.zeros((SEQ_LEN, D_MODEL), dtype=np.float32)
    pe[:, 0::2] = np.sin(pos * div)
    pe[:, 1::2] = np.cos(pos * div)
    return pe


_PE = jnp.asarray(_make_pe())

_mesh = plsc.VectorSubcoreMesh(core_axis_name="c", subcore_axis_name="s")


@functools.partial(
    pl.kernel,
    mesh=_mesh,
    out_type=jax.ShapeDtypeStruct((BATCH * SEQ_LEN, D_MODEL), jnp.float32),
    scratch_types=[
        pltpu.VMEM((SEQ_PER_W,), jnp.int32),
        pltpu.VMEM((SEQ_PER_W, D_MODEL), jnp.float32),
        pltpu.VMEM((SEQ_PER_W, D_MODEL), jnp.float32),
        pltpu.SemaphoreType.DMA,
    ],
)
def _encode(tokens_hbm, pe_hbm, table_hbm, out_hbm, idx_v, pe_v, rows_v, sem):
    wid = lax.axis_index("s") * 2 + lax.axis_index("c")
    seq_base = wid * SEQ_PER_W
    pltpu.sync_copy(pe_hbm.at[pl.ds(seq_base, SEQ_PER_W)], pe_v)
    for b in range(BATCH):
        flat = b * SEQ_LEN + seq_base
        pltpu.sync_copy(tokens_hbm.at[pl.ds(flat, SEQ_PER_W)], idx_v)
        pltpu.async_copy(table_hbm.at[idx_v], rows_v, sem).wait()

        def body(r, carry):
            for c in range(CHUNKS_PER_ROW):
                sl = pl.ds(c * LANES, LANES)
                rows_v[r, sl] = rows_v[r, sl] * SCALE + pe_v[r, sl]
            return carry

        lax.fori_loop(0, SEQ_PER_W, body, 0)
        pltpu.sync_copy(rows_v, out_hbm.at[pl.ds(flat, SEQ_PER_W)])


def kernel(tokens, table):
    tokens_flat = tokens.reshape(-1).astype(jnp.int32)
    out = _encode(tokens_flat, _PE, table)
    return out.reshape(BATCH, SEQ_LEN, D_MODEL)
